# Initial kernel scaffold; baseline (speedup 1.0000x reference)
import jax
import jax.numpy as jnp
from jax.experimental import pallas as pl


def kernel(objectness, box_regression, anchors):
    raise NotImplementedError("write your pallas kernel here")


# speedup vs baseline: 24.2826x; 24.2826x over previous
import numpy as np
import jax
import jax.numpy as jnp
from jax.experimental import pallas as pl
from jax.experimental.pallas import tpu as pltpu

_PRE = 2000
_POST = 1000
_NMS_T = 0.7
_IMW = 800.0
_IMH = 800.0
_CLIP = float(np.log(1000.0 / 16.0))
_K = 30000
_NCH = 15
_CH = _K // _NCH
_RB = 200
_I32MIN = np.int32(-2147483648)


def _sortable(x):
    s = jax.lax.bitcast_convert_type(x, jnp.int32)
    return s ^ (jax.lax.shift_right_arithmetic(s, 31) & jnp.int32(0x7FFFFFFF))


def _split_bytes(x):
    k = jax.lax.bitcast_convert_type(x, jnp.int32)
    planes = [
        (jax.lax.shift_right_logical(k, 8 * i) & jnp.int32(255)).astype(jnp.float32)
        for i in range(4)
    ]
    return jnp.concatenate(planes, axis=0)


def _join_bytes(b):
    m = b.shape[0] // 4
    k = b[0:m].astype(jnp.int32)
    for i in range(1, 4):
        k = k | jax.lax.shift_left(b[i * m:(i + 1) * m].astype(jnp.int32), 8 * i)
    return jax.lax.bitcast_convert_type(k, jnp.float32)


def _tri_le():
    a = jax.lax.broadcasted_iota(jnp.int32, (_CH, _CH), 0)
    b = jax.lax.broadcasted_iota(jnp.int32, (_CH, _CH), 1)
    return (a <= b).astype(jnp.float32)


def _cumsum_chunks(mask_f, lt):
    y = jnp.dot(mask_f, lt, preferred_element_type=jnp.float32)
    tot = y[:, _CH - 1:_CH]
    r0 = jax.lax.broadcasted_iota(jnp.int32, (_NCH, _NCH), 0)
    r1 = jax.lax.broadcasted_iota(jnp.int32, (_NCH, _NCH), 1)
    strictly_before = (r0 < r1).astype(jnp.float32)
    offs = jnp.sum(tot * strictly_before, axis=0)
    return y + offs[:, None]


def _body(logit_ref, reg_ref, anc_ref, boxes_ref, score_ref, s_ref):
    lg = logit_ref[0]
    key = _sortable(lg)

    cnt_pos = jnp.sum((key >= 0).astype(jnp.float32))
    thr = jnp.where(cnt_pos >= float(_PRE), jnp.int32(0), _I32MIN)
    for b in range(30, -1, -1):
        cand = thr | jnp.int32(1 << b)
        cnt = jnp.sum((key >= cand).astype(jnp.float32))
        thr = jnp.where(cnt >= float(_PRE), cand, thr)

    gt = key > thr
    eq = key == thr
    n_gt = jnp.sum(gt.astype(jnp.float32))
    ties_budget = float(_PRE) - n_gt

    lt = _tri_le()
    cume = _cumsum_chunks(eq.astype(jnp.float32), lt)
    sel = jnp.logical_or(gt, jnp.logical_and(eq, cume <= ties_budget))
    self_f = sel.astype(jnp.float32)
    dest = _cumsum_chunks(self_f, lt) - 1.0

    slot_iota = jax.lax.broadcasted_iota(jnp.int32, (_CH, _PRE), 1)
    acc = jnp.zeros((36, _PRE), dtype=jnp.float32)
    reg = reg_ref[0]
    anc = anc_ref[0]
    for c in range(_NCH):
        dcol = jnp.transpose(jnp.where(sel[c], dest[c], -1.0)[None, :])
        onehot = (dcol.astype(jnp.int32) == slot_iota).astype(jnp.float32)
        vals = jnp.concatenate(
            [lg[c:c + 1], reg[:, c * _CH:(c + 1) * _CH], anc[:, c * _CH:(c + 1) * _CH]],
            axis=0,
        )
        acc = acc + jnp.dot(_split_bytes(vals), onehot,
                            preferred_element_type=jnp.float32)

    lrow = _join_bytes(jnp.concatenate(
        [acc[0:1], acc[9:10], acc[18:19], acc[27:28]], axis=0))
    krow = _sortable(lrow)
    kcol = _sortable(jnp.transpose(lrow))
    ciota = jax.lax.broadcasted_iota(jnp.int32, (_RB, _PRE), 1)
    riota = jax.lax.broadcasted_iota(jnp.int32, (_RB, _PRE), 0)
    rank = jnp.zeros((1, _PRE), dtype=jnp.float32)
    for rb in range(_PRE // _RB):
        kb = kcol[rb * _RB:(rb + 1) * _RB]
        prec = jnp.logical_or(
            kb > krow,
            jnp.logical_and(kb == krow, riota + rb * _RB < ciota),
        )
        rank = rank + jnp.sum(prec.astype(jnp.float32), axis=0)[None, :]
    rcol = jnp.transpose(rank)
    rt = (rcol.astype(jnp.int32) == slot_iota).astype(jnp.float32)
    srt = _join_bytes(jnp.dot(acc, rt, preferred_element_type=jnp.float32))

    top_scores = jax.nn.sigmoid(srt[0])
    dx, dy = srt[1], srt[2]
    dw = jnp.minimum(srt[3], _CLIP)
    dh = jnp.minimum(srt[4], _CLIP)
    aw = srt[7] - srt[5] + 1.0
    ah = srt[8] - srt[6] + 1.0
    cx = srt[5] + 0.5 * aw
    cy = srt[6] + 0.5 * ah
    px = dx * aw + cx
    py = dy * ah + cy
    pw = jnp.exp(dw) * aw
    ph = jnp.exp(dh) * ah
    x1 = jnp.clip(px - 0.5 * pw, 0.0, _IMW - 1.0)
    y1 = jnp.clip(py - 0.5 * ph, 0.0, _IMH - 1.0)
    x2 = jnp.clip(px + 0.5 * pw - 1.0, 0.0, _IMW - 1.0)
    y2 = jnp.clip(py + 0.5 * ph - 1.0, 0.0, _IMH - 1.0)
    ws = x2 - x1 + 1.0
    hs = y2 - y1 + 1.0
    valid = (ws >= 0.0) & (hs >= 0.0)
    area = ws * hs

    for b in range(_PRE // _RB):
        r0 = b * _RB
        bx1 = x1[r0:r0 + _RB][:, None]
        by1 = y1[r0:r0 + _RB][:, None]
        bx2 = x2[r0:r0 + _RB][:, None]
        by2 = y2[r0:r0 + _RB][:, None]
        barea = area[r0:r0 + _RB][:, None]
        iw = jnp.clip(jnp.minimum(bx2, x2[None, :]) - jnp.maximum(bx1, x1[None, :]) + 1.0, 0.0)
        ih = jnp.clip(jnp.minimum(by2, y2[None, :]) - jnp.maximum(by1, y1[None, :]) + 1.0, 0.0)
        inter = iw * ih
        iou = inter / (barea + area[None, :] - inter)
        s = (iou > _NMS_T) & (riota + r0 < ciota)
        s_ref[r0:r0 + _RB, :] = s.astype(jnp.float32)

    validf = valid.astype(jnp.float32)[None, :]

    def nms_cond(carry):
        keep, prev, it = carry
        return jnp.logical_and(it < _PRE, jnp.any(keep != prev))

    def nms_step(carry):
        keep, _, it = carry
        sup = jnp.dot(keep, s_ref[...], preferred_element_type=jnp.float32)
        new = validf * (sup < 0.5).astype(jnp.float32)
        return new, keep, it + 1

    keep, _, _ = jax.lax.while_loop(
        nms_cond, nms_step, (validf, -jnp.ones_like(validf), jnp.int32(0))
    )

    ics = jnp.dot(keep, lt, preferred_element_type=jnp.float32)
    fdest = ics - 1.0
    fdest = jnp.where((keep > 0.5) & (fdest < float(_POST)), fdest, -1.0)
    fcol = jnp.transpose(fdest)
    out_iota = jax.lax.broadcasted_iota(jnp.int32, (_PRE, _POST), 1)
    oh = (fcol.astype(jnp.int32) == out_iota).astype(jnp.float32)
    vals5 = jnp.stack([top_scores, x1, y1, x2, y2], axis=0)
    fin = _join_bytes(jnp.dot(_split_bytes(vals5), oh,
                              preferred_element_type=jnp.float32))
    score_ref[0] = fin[0:1]
    boxes_ref[0] = fin[1:5]


def kernel(objectness, box_regression, anchors):
    N, A, H, W = objectness.shape
    K = A * H * W
    logits = objectness.transpose(0, 2, 3, 1).reshape(N, _NCH, _CH)
    reg = box_regression.reshape(N, A, 4, H, W).transpose(0, 2, 3, 4, 1).reshape(N, 4, K)
    anc = anchors.transpose(0, 2, 1)

    boxes_cm, scores = pl.pallas_call(
        _body,
        grid=(N,),
        in_specs=[
            pl.BlockSpec((1, _NCH, _CH), lambda i: (i, 0, 0)),
            pl.BlockSpec((1, 4, K), lambda i: (i, 0, 0)),
            pl.BlockSpec((1, 4, K), lambda i: (i, 0, 0)),
        ],
        out_specs=[
            pl.BlockSpec((1, 4, _POST), lambda i: (i, 0, 0)),
            pl.BlockSpec((1, 1, _POST), lambda i: (i, 0, 0)),
        ],
        out_shape=[
            jax.ShapeDtypeStruct((N, 4, _POST), jnp.float32),
            jax.ShapeDtypeStruct((N, 1, _POST), jnp.float32),
        ],
        scratch_shapes=[pltpu.VMEM((_PRE, _PRE), jnp.float32)],
    )(logits, reg, anc)
    return boxes_cm.transpose(0, 2, 1), scores.reshape(N, _POST)
